# baseline (speedup 1.0000x reference)
import functools

import jax
import jax.numpy as jnp
from jax import lax
from jax.experimental import pallas as pl
from jax.experimental.pallas import tpu as pltpu
from jax.experimental.pallas import tpu_sc as plsc

N_NODES = 10000
N_EDGES = 320000
D_FEAT = 128

NC = 2
NS = 16
NW = NC * NS
CHUNK = 128
NCHUNKS = 80
EDGES_PER_TILE = NCHUNKS * CHUNK
PADDED_E = NW * EDGES_PER_TILE
NHALF = 2
CHUNKS_PER_HALF = NCHUNKS // NHALF
ACC_ROWS = 10240
ROWS_PER_TILE = ACC_ROWS // NS
N_PAD_ROWS = ACC_ROWS - N_NODES


NBUF = 2


def _sc_kernel_body(x_hbm, s3_hbm, r3_hbm, zeros_hbm, out_hbm,
                    sidx_v, ridx_v, rows_v, sems, zsem, acc_sh):
    c = lax.axis_index("c")
    s = lax.axis_index("s")
    tid = c * NS + s
    row0 = s * ROWS_PER_TILE

    zero_cp = pltpu.async_copy(zeros_hbm, acc_sh.at[pl.ds(row0, ROWS_PER_TILE)],
                               zsem)
    pltpu.sync_copy(s3_hbm.at[tid, 0], sidx_v)
    pltpu.sync_copy(r3_hbm.at[tid, 0], ridx_v)
    for b in range(NBUF):
        pltpu.async_copy(x_hbm.at[sidx_v.at[b]], rows_v[b], sems[b])
    zero_cp.wait()
    plsc.subcore_barrier()

    for h in range(NHALF):
        if h > 0:
            pltpu.sync_copy(s3_hbm.at[tid, h], sidx_v)
            pltpu.sync_copy(r3_hbm.at[tid, h], ridx_v)
            for b in range(NBUF):
                pltpu.async_copy(x_hbm.at[sidx_v.at[b]], rows_v[b], sems[b])

        def chunk_body(g, _):
            for b in range(NBUF):
                j = g * NBUF + b
                pltpu.make_async_copy(x_hbm.at[sidx_v.at[j]], rows_v[b],
                                      sems[b]).wait()
                pltpu.sync_copy(rows_v[b], acc_sh.at[ridx_v.at[j]], add=True)

                @pl.when(j + NBUF < CHUNKS_PER_HALF)
                def _issue():
                    pltpu.async_copy(x_hbm.at[sidx_v.at[j + NBUF]], rows_v[b],
                                     sems[b])
            return _

        lax.fori_loop(0, CHUNKS_PER_HALF // NBUF, chunk_body, None)

    plsc.subcore_barrier()

    pltpu.sync_copy(acc_sh.at[pl.ds(row0, ROWS_PER_TILE)],
                    out_hbm.at[c, pl.ds(row0, ROWS_PER_TILE)])


_sc_call = functools.partial(
    pl.kernel,
    out_type=jax.ShapeDtypeStruct((NC, ACC_ROWS, D_FEAT), jnp.float32),
    mesh=plsc.VectorSubcoreMesh(core_axis_name="c", subcore_axis_name="s"),
    scratch_types=[
        pltpu.VMEM((CHUNKS_PER_HALF, CHUNK), jnp.int32),
        pltpu.VMEM((CHUNKS_PER_HALF, CHUNK), jnp.int32),
        [pltpu.VMEM((CHUNK, D_FEAT), jnp.float32)] * NBUF,
        [pltpu.SemaphoreType.DMA] * NBUF,
        pltpu.SemaphoreType.DMA,
        pltpu.VMEM_SHARED((ACC_ROWS, D_FEAT), jnp.float32),
    ],
)(_sc_kernel_body)


TC_BLOCK = 1000


def _tc_kernel_body(p_ref, w_ref, o_ref):
    summed = p_ref[0] + p_ref[1]
    o_ref[...] = jnp.maximum(
        jax.lax.dot(summed, w_ref[...], preferred_element_type=jnp.float32), 0.0)


def _tc_matmul(partials, W):
    return pl.pallas_call(
        _tc_kernel_body,
        grid=(N_NODES // TC_BLOCK,),
        in_specs=[
            pl.BlockSpec((NC, TC_BLOCK, D_FEAT), lambda i: (0, i, 0)),
            pl.BlockSpec((D_FEAT, D_FEAT), lambda i: (0, 0)),
        ],
        out_specs=pl.BlockSpec((TC_BLOCK, D_FEAT), lambda i: (i, 0)),
        out_shape=jax.ShapeDtypeStruct((N_NODES, D_FEAT), jnp.float32),
    )(partials, W)


def kernel(x, edge_index, W):
    pad = PADDED_E - N_EDGES
    sender = jnp.concatenate(
        [edge_index[0].astype(jnp.int32), jnp.zeros((pad,), jnp.int32)]
    ).reshape(NW, NHALF, CHUNKS_PER_HALF, CHUNK)
    pad_recv = N_NODES + (jnp.arange(pad, dtype=jnp.int32) % N_PAD_ROWS)
    receiver = jnp.concatenate(
        [edge_index[1].astype(jnp.int32), pad_recv]
    ).reshape(NW, NHALF, CHUNKS_PER_HALF, CHUNK)
    zeros = jnp.zeros((ROWS_PER_TILE, D_FEAT), jnp.float32)
    partials = _sc_call(x, sender, receiver, zeros)
    return _tc_matmul(partials, W)


# speedup vs baseline: 3.4929x; 3.4929x over previous
import functools

import jax
import jax.numpy as jnp
from jax import lax
from jax.experimental import pallas as pl
from jax.experimental.pallas import tpu as pltpu
from jax.experimental.pallas import tpu_sc as plsc

N_NODES = 10000
N_EDGES = 320000
D_FEAT = 128

NC = 2
NS = 16
NW = NC * NS
CHUNK = 100
NCHUNKS = 100
EDGES_PER_TILE = NCHUNKS * CHUNK
NHALF = 2
CHUNKS_PER_HALF = NCHUNKS // NHALF
ACC_ROWS = 10240
ROWS_PER_TILE = ACC_ROWS // NS


NBUF = 2


def _sc_kernel_body(x_hbm, s3_hbm, r3_hbm, zeros_hbm, out_hbm,
                    sidx_v, ridx_v, rows_v, sems, zsem, acc_sh):
    c = lax.axis_index("c")
    s = lax.axis_index("s")
    tid = c * NS + s
    row0 = s * ROWS_PER_TILE

    zero_cp = pltpu.async_copy(zeros_hbm, acc_sh.at[pl.ds(row0, ROWS_PER_TILE)],
                               zsem)
    pltpu.sync_copy(s3_hbm.at[tid, 0], sidx_v)
    pltpu.sync_copy(r3_hbm.at[tid, 0], ridx_v)
    for b in range(NBUF):
        pltpu.async_copy(x_hbm.at[sidx_v.at[b]], rows_v[b], sems[b])
    zero_cp.wait()
    plsc.subcore_barrier()

    for h in range(NHALF):
        if h > 0:
            pltpu.sync_copy(s3_hbm.at[tid, h], sidx_v)
            pltpu.sync_copy(r3_hbm.at[tid, h], ridx_v)
            for b in range(NBUF):
                pltpu.async_copy(x_hbm.at[sidx_v.at[b]], rows_v[b], sems[b])

        def chunk_body(g, _):
            for b in range(NBUF):
                j = g * NBUF + b
                pltpu.make_async_copy(x_hbm.at[sidx_v.at[j]], rows_v[b],
                                      sems[b]).wait()
                pltpu.sync_copy(rows_v[b], acc_sh.at[ridx_v.at[j]], add=True)

                @pl.when(j + NBUF < CHUNKS_PER_HALF)
                def _issue():
                    pltpu.async_copy(x_hbm.at[sidx_v.at[j + NBUF]], rows_v[b],
                                     sems[b])
            return _

        lax.fori_loop(0, CHUNKS_PER_HALF // NBUF, chunk_body, None)

    plsc.subcore_barrier()

    pltpu.sync_copy(acc_sh.at[pl.ds(row0, ROWS_PER_TILE)],
                    out_hbm.at[c, pl.ds(row0, ROWS_PER_TILE)])


_sc_call = functools.partial(
    pl.kernel,
    out_type=jax.ShapeDtypeStruct((NC, ACC_ROWS, D_FEAT), jnp.float32),
    mesh=plsc.VectorSubcoreMesh(core_axis_name="c", subcore_axis_name="s"),
    scratch_types=[
        pltpu.VMEM((CHUNKS_PER_HALF, CHUNK), jnp.int32),
        pltpu.VMEM((CHUNKS_PER_HALF, CHUNK), jnp.int32),
        [pltpu.VMEM((CHUNK, D_FEAT), jnp.float32)] * NBUF,
        [pltpu.SemaphoreType.DMA] * NBUF,
        pltpu.SemaphoreType.DMA,
        pltpu.VMEM_SHARED((ACC_ROWS, D_FEAT), jnp.float32),
    ],
)(_sc_kernel_body)


TC_BLOCK = 1000


def _tc_kernel_body(p_ref, w_ref, o_ref):
    summed = p_ref[0] + p_ref[1]
    o_ref[...] = jnp.maximum(
        jax.lax.dot(summed, w_ref[...], preferred_element_type=jnp.float32), 0.0)


def _tc_matmul(partials, W):
    return pl.pallas_call(
        _tc_kernel_body,
        grid=(N_NODES // TC_BLOCK,),
        in_specs=[
            pl.BlockSpec((NC, TC_BLOCK, D_FEAT), lambda i: (0, i, 0)),
            pl.BlockSpec((D_FEAT, D_FEAT), lambda i: (0, 0)),
        ],
        out_specs=pl.BlockSpec((TC_BLOCK, D_FEAT), lambda i: (i, 0)),
        out_shape=jax.ShapeDtypeStruct((N_NODES, D_FEAT), jnp.float32),
    )(partials, W)


def kernel(x, edge_index, W):
    sender = edge_index[0].astype(jnp.int32).reshape(
        NW, NHALF, CHUNKS_PER_HALF, CHUNK)
    receiver = edge_index[1].astype(jnp.int32).reshape(
        NW, NHALF, CHUNKS_PER_HALF, CHUNK)
    zeros = jnp.zeros((ROWS_PER_TILE, D_FEAT), jnp.float32)
    partials = _sc_call(x, sender, receiver, zeros)
    return _tc_matmul(partials, W)
